# baseline (speedup 1.0000x reference)
import functools

import jax
import jax.numpy as jnp
from jax import lax
from jax.experimental import pallas as pl
from jax.experimental.pallas import tpu as pltpu
from jax.experimental.pallas import tpu_sc as plsc


def _worker_id():
    return lax.axis_index("s") * 2 + lax.axis_index("c")


def _iota16():
    return lax.iota(jnp.int32, 16)


def _make_transpose_pack(Dt, V):
    NW = 32
    n_win = V // 128
    n_rem = V - n_win * 128
    base, rem = n_win // NW, n_win % NW
    NS = 3
    n_groups = (base + (1 if rem else 0) + NS - 1) // NS

    mesh = plsc.VectorSubcoreMesh(core_axis_name="c", subcore_axis_name="s")

    @functools.partial(
        pl.kernel,
        mesh=mesh,
        compiler_params=pltpu.CompilerParams(needs_layout_passes=False),
        out_type=jax.ShapeDtypeStruct((V // 2, 128), jnp.float32),
        scratch_types=[
            [pltpu.VMEM((Dt, 128), jnp.float32) for _ in range(3)],
            [pltpu.VMEM((Dt, 128), jnp.float32) for _ in range(3)],
            [pltpu.SemaphoreType.DMA for _ in range(3)],
            [pltpu.SemaphoreType.DMA for _ in range(3)],
        ],
    )
    def k(tab_hbm, tail_hbm, scr_hbm, inb, tbuf, sem_i, sem_w):
        t = _worker_id()
        nt = base + jnp.where(t < rem, 1, 0)

        def win_off(kk):
            return pl.multiple_of((t + NW * kk) * 128, 128)

        def instage(kk, p):
            pltpu.async_copy(
                tab_hbm.at[:, pl.ds(win_off(kk), 128)], inb[p], sem_i[p]
            )

        instage(0, 0)
        for q in (1, 2):
            @pl.when(q < nt)
            def _():
                instage(q, q)

        iota = _iota16()
        iota_half = iota >> 1
        iota_par64 = (iota & 1) * 64

        def body(grp, _):
            for p in range(NS):
                kk = grp * NS + p

                @pl.when(kk < nt)
                def _():
                    pltpu.make_async_copy(
                        tab_hbm.at[:, pl.ds(0, 128)], inb[p], sem_i[p]
                    ).wait()

                    @pl.when(kk >= NS)
                    def _():
                        pltpu.make_async_copy(
                            tbuf[p], scr_hbm.at[pl.ds(0, Dt)], sem_w[p]
                        ).wait()

                    @plsc.parallel_loop(0, (Dt // 16) * 8, unroll=8)
                    def trans(blk):
                        d0 = (blk >> 3) * 16
                        m0 = (blk & 7) * 16
                        drow = (m0 >> 1) + iota_half
                        mcol = m0 + iota
                        for j in range(16):
                            srow = d0 + ((iota + j) & 15)
                            vals = plsc.load_gather(inb[p], [srow, mcol])
                            plsc.store_scatter(
                                tbuf[p], [drow, iota_par64 + srow], vals
                            )
                    pltpu.async_copy(
                        tbuf[p],
                        scr_hbm.at[pl.ds(pl.multiple_of(win_off(kk) >> 1, 64), Dt)],
                        sem_w[p],
                    )

                    @pl.when(kk + NS < nt)
                    def _():
                        instage(kk + NS, p)

            return _

        lax.fori_loop(0, n_groups, body, None)
        for p in range(NS):
            @pl.when(p < nt)
            def _():
                pltpu.make_async_copy(
                    tbuf[p], scr_hbm.at[pl.ds(0, Dt)], sem_w[p]
                ).wait()

        if n_rem:
            @pl.when(t == NW - 1)
            def _():
                pltpu.sync_copy(tail_hbm, inb[0].at[pl.ds(0, n_rem // 2), :])
                pltpu.sync_copy(
                    inb[0].at[pl.ds(0, n_rem // 2), :],
                    scr_hbm.at[pl.ds(n_win * 64, n_rem // 2)],
                )

    return k


def _make_gather_assemble(S, Bt, D, V):
    NW = 32
    NBT = Bt // 128
    n_units = S * NBT
    upw = n_units // NW
    assert upw % 2 == 0

    mesh = plsc.VectorSubcoreMesh(core_axis_name="c", subcore_axis_name="s")

    @functools.partial(
        pl.kernel,
        mesh=mesh,
        compiler_params=pltpu.CompilerParams(needs_layout_passes=False),
        out_type=jax.ShapeDtypeStruct((S, D, Bt), jnp.float32),
        scratch_types=[
            [pltpu.VMEM((8, 128), jnp.int32) for _ in range(2)],
            [pltpu.VMEM((128,), jnp.int32) for _ in range(2)],
            [pltpu.VMEM((128,), jnp.int32) for _ in range(2)],
            [pltpu.VMEM((128, 128), jnp.float32) for _ in range(2)],
            [pltpu.VMEM((D, 128), jnp.float32) for _ in range(2)],
            [pltpu.SemaphoreType.DMA for _ in range(2)],
            [pltpu.SemaphoreType.DMA for _ in range(2)],
            [pltpu.SemaphoreType.DMA for _ in range(2)],
        ],
    )
    def k(scr_hbm, idx_hbm, out_hbm, idxb, rowb, parb, gbuf, tbuf,
          sem_i, sem_g, sem_w):
        t = _worker_id()
        iota = _iota16()

        def unit(kk):
            u = t * upw + kk
            return u // NBT, pl.multiple_of((u % NBT) * 128, 128)

        def idxstage(kk, p):
            s, b0 = unit(kk)
            sblk = pl.multiple_of((s // 8) * 8, 8)
            pltpu.async_copy(
                idx_hbm.at[pl.ds(sblk, 8), pl.ds(b0, 128)], idxb[p], sem_i[p]
            )

        def assemble_and_write(kk, p):
            @plsc.parallel_loop(0, 8 * (D // 16), unroll=8)
            def arow(blk):
                l0 = (blk >> 2) * 16
                d0 = (blk & 3) * 16
                rows16 = l0 + iota
                cols_base = parb[p][pl.ds(l0, 16)] + d0
                for j in range(16):
                    perm = (iota + j) & 15
                    vals = plsc.load_gather(gbuf[p], [rows16, cols_base + perm])
                    plsc.store_scatter(tbuf[p], [d0 + perm, rows16], vals)
            s, b0 = unit(kk)
            pltpu.async_copy(
                tbuf[p], out_hbm.at[s, :, pl.ds(b0, 128)], sem_w[p]
            )

        idxstage(0, 0)

        def body(grp, _):
            for p in range(2):
                kk = grp * 2 + p
                p1 = 1 - p
                pltpu.make_async_copy(
                    idx_hbm.at[pl.ds(0, 8), pl.ds(0, 128)], idxb[p], sem_i[p]
                ).wait()
                s, _b0 = unit(kk)
                soff = s % 8
                for g in range(8):
                    v = idxb[p][soff, pl.ds(g * 16, 16)]
                    rowb[p][pl.ds(g * 16, 16)] = v >> 1
                    parb[p][pl.ds(g * 16, 16)] = (v & 1) * 64

                @pl.when(kk + 1 < upw)
                def _():
                    idxstage(kk + 1, p1)

                pltpu.async_copy(scr_hbm.at[rowb[p]], gbuf[p], sem_g[p])

                @pl.when(kk >= 1)
                def _():
                    pltpu.make_async_copy(
                        scr_hbm.at[rowb[p1]], gbuf[p1], sem_g[p1]
                    ).wait()

                    @pl.when(kk >= 3)
                    def _():
                        pltpu.make_async_copy(
                            tbuf[p1], out_hbm.at[0, :, pl.ds(0, 128)],
                            sem_w[p1],
                        ).wait()

                    assemble_and_write(kk - 1, p1)
            return _

        lax.fori_loop(0, upw // 2, body, None)
        pltpu.make_async_copy(
            scr_hbm.at[rowb[1]], gbuf[1], sem_g[1]
        ).wait()
        pltpu.make_async_copy(
            tbuf[1], out_hbm.at[0, :, pl.ds(0, 128)], sem_w[1]
        ).wait()
        assemble_and_write(upw - 1, 1)
        for p in range(2):
            pltpu.make_async_copy(
                tbuf[p], out_hbm.at[0, :, pl.ds(0, 128)], sem_w[p]
            ).wait()

    return k


def kernel(inputs, table):
    S, Bt = inputs.shape
    V, D = table.shape
    n_win = V // 128
    tail = table[n_win * 128:].reshape(-1, 128)
    scratch = _make_transpose_pack(D, V)(table.T, tail)
    outT = _make_gather_assemble(S, Bt, D, V)(scratch, inputs)
    return outT.transpose(0, 2, 1)


# speedup vs baseline: 1.2954x; 1.2954x over previous
import functools

import jax
import jax.numpy as jnp
from jax import lax
from jax.experimental import pallas as pl
from jax.experimental.pallas import tpu as pltpu
from jax.experimental.pallas import tpu_sc as plsc


def _worker_id():
    return lax.axis_index("s") * 2 + lax.axis_index("c")


def _iota16():
    return lax.iota(jnp.int32, 16)


def _make_transpose_pack(Dt, V):
    NW = 32
    n_win = V // 128
    n_rem = V - n_win * 128
    base, rem = n_win // NW, n_win % NW
    n_groups = (base + (1 if rem else 0) + 1) // 2

    mesh = plsc.VectorSubcoreMesh(core_axis_name="c", subcore_axis_name="s")

    @functools.partial(
        pl.kernel,
        mesh=mesh,
        compiler_params=pltpu.CompilerParams(needs_layout_passes=False),
        out_type=jax.ShapeDtypeStruct((V // 2, 128), jnp.float32),
        scratch_types=[
            [pltpu.VMEM((Dt, 128), jnp.float32) for _ in range(2)],
            [pltpu.VMEM((Dt, 128), jnp.float32) for _ in range(2)],
            [pltpu.SemaphoreType.DMA for _ in range(2)],
            [pltpu.SemaphoreType.DMA for _ in range(2)],
        ],
    )
    def k(tab_hbm, tail_hbm, scr_hbm, inb, tbuf, sem_i, sem_w):
        t = _worker_id()
        nt = base + jnp.where(t < rem, 1, 0)

        def win_off(kk):
            return pl.multiple_of((t + NW * kk) * 128, 128)

        def instage(kk, p):
            pltpu.async_copy(
                tab_hbm.at[:, pl.ds(win_off(kk), 128)], inb[p], sem_i[p]
            )

        instage(0, 0)

        @pl.when(1 < nt)
        def _():
            instage(1, 1)

        iota = _iota16()
        iota_half = iota >> 1
        iota_par64 = (iota & 1) * 64

        def body(grp, _):
            for p in range(2):
                kk = grp * 2 + p

                @pl.when(kk < nt)
                def _():
                    pltpu.make_async_copy(
                        tab_hbm.at[:, pl.ds(0, 128)], inb[p], sem_i[p]
                    ).wait()

                    @pl.when(kk >= 2)
                    def _():
                        pltpu.make_async_copy(
                            tbuf[p], scr_hbm.at[pl.ds(0, Dt)], sem_w[p]
                        ).wait()

                    @plsc.parallel_loop(0, (Dt // 16) * 8, unroll=4)
                    def trans(blk):
                        d0 = (blk >> 3) * 16
                        m0 = (blk & 7) * 16
                        drow = (m0 >> 1) + iota_half
                        mcol = m0 + iota
                        for j in range(16):
                            srow = d0 + ((iota + j) & 15)
                            vals = plsc.load_gather(inb[p], [srow, mcol])
                            plsc.store_scatter(
                                tbuf[p], [drow, iota_par64 + srow], vals
                            )
                    pltpu.async_copy(
                        tbuf[p],
                        scr_hbm.at[pl.ds(pl.multiple_of(win_off(kk) >> 1, 64), Dt)],
                        sem_w[p],
                    )

                    @pl.when(kk + 2 < nt)
                    def _():
                        instage(kk + 2, p)

            return _

        lax.fori_loop(0, n_groups, body, None)
        for p in range(2):
            pltpu.make_async_copy(
                tbuf[p], scr_hbm.at[pl.ds(0, Dt)], sem_w[p]
            ).wait()

        if n_rem:
            @pl.when(t == NW - 1)
            def _():
                pltpu.sync_copy(tail_hbm, inb[0].at[pl.ds(0, n_rem // 2), :])
                pltpu.sync_copy(
                    inb[0].at[pl.ds(0, n_rem // 2), :],
                    scr_hbm.at[pl.ds(n_win * 64, n_rem // 2)],
                )

    return k


def _make_gather_assemble(S, Bt, D, V):
    NW = 32
    NBT = Bt // 128
    n_units = S * NBT
    upw = n_units // NW
    assert upw % 2 == 0

    mesh = plsc.VectorSubcoreMesh(core_axis_name="c", subcore_axis_name="s")

    @functools.partial(
        pl.kernel,
        mesh=mesh,
        compiler_params=pltpu.CompilerParams(needs_layout_passes=False),
        out_type=jax.ShapeDtypeStruct((S, D, Bt), jnp.float32),
        scratch_types=[
            [pltpu.VMEM((8, 128), jnp.int32) for _ in range(2)],
            [pltpu.VMEM((128,), jnp.int32) for _ in range(2)],
            [pltpu.VMEM((128,), jnp.int32) for _ in range(2)],
            [pltpu.VMEM((128, 128), jnp.float32) for _ in range(2)],
            [pltpu.VMEM((D, 128), jnp.float32) for _ in range(2)],
            [pltpu.SemaphoreType.DMA for _ in range(2)],
            [pltpu.SemaphoreType.DMA for _ in range(2)],
            [pltpu.SemaphoreType.DMA for _ in range(2)],
        ],
    )
    def k(scr_hbm, idx_hbm, out_hbm, idxb, rowb, parb, gbuf, tbuf,
          sem_i, sem_g, sem_w):
        t = _worker_id()
        iota = _iota16()

        def unit(kk):
            u = t * upw + kk
            return u // NBT, pl.multiple_of((u % NBT) * 128, 128)

        def idxstage(kk, p):
            s, b0 = unit(kk)
            sblk = pl.multiple_of((s // 8) * 8, 8)
            pltpu.async_copy(
                idx_hbm.at[pl.ds(sblk, 8), pl.ds(b0, 128)], idxb[p], sem_i[p]
            )

        def assemble_and_write(kk, p):
            @plsc.parallel_loop(0, 8 * (D // 16), unroll=4)
            def arow(blk):
                l0 = (blk >> 2) * 16
                d0 = (blk & 3) * 16
                rows16 = l0 + iota
                cols_base = parb[p][pl.ds(l0, 16)] + d0
                for j in range(16):
                    perm = (iota + j) & 15
                    vals = plsc.load_gather(gbuf[p], [rows16, cols_base + perm])
                    plsc.store_scatter(tbuf[p], [d0 + perm, rows16], vals)
            s, b0 = unit(kk)
            pltpu.async_copy(
                tbuf[p], out_hbm.at[s, :, pl.ds(b0, 128)], sem_w[p]
            )

        idxstage(0, 0)

        def body(grp, _):
            for p in range(2):
                kk = grp * 2 + p
                p1 = 1 - p
                pltpu.make_async_copy(
                    idx_hbm.at[pl.ds(0, 8), pl.ds(0, 128)], idxb[p], sem_i[p]
                ).wait()
                s, _b0 = unit(kk)
                soff = s % 8
                for g in range(8):
                    v = idxb[p][soff, pl.ds(g * 16, 16)]
                    rowb[p][pl.ds(g * 16, 16)] = v >> 1
                    parb[p][pl.ds(g * 16, 16)] = (v & 1) * 64

                @pl.when(kk + 1 < upw)
                def _():
                    idxstage(kk + 1, p1)

                pltpu.async_copy(scr_hbm.at[rowb[p]], gbuf[p], sem_g[p])

                @pl.when(kk >= 1)
                def _():
                    pltpu.make_async_copy(
                        scr_hbm.at[rowb[p1]], gbuf[p1], sem_g[p1]
                    ).wait()

                    @pl.when(kk >= 3)
                    def _():
                        pltpu.make_async_copy(
                            tbuf[p1], out_hbm.at[0, :, pl.ds(0, 128)],
                            sem_w[p1],
                        ).wait()

                    assemble_and_write(kk - 1, p1)
            return _

        lax.fori_loop(0, upw // 2, body, None)
        pltpu.make_async_copy(
            scr_hbm.at[rowb[1]], gbuf[1], sem_g[1]
        ).wait()
        pltpu.make_async_copy(
            tbuf[1], out_hbm.at[0, :, pl.ds(0, 128)], sem_w[1]
        ).wait()
        assemble_and_write(upw - 1, 1)
        for p in range(2):
            pltpu.make_async_copy(
                tbuf[p], out_hbm.at[0, :, pl.ds(0, 128)], sem_w[p]
            ).wait()

    return k


def kernel(inputs, table):
    S, Bt = inputs.shape
    V, D = table.shape
    n_win = V // 128
    tail = table[n_win * 128:].reshape(-1, 128)
    scratch = _make_transpose_pack(D, V)(table.T, tail)
    outT = _make_gather_assemble(S, Bt, D, V)(scratch, inputs)
    return outT.transpose(0, 2, 1)
